# Initial kernel scaffold; baseline (speedup 1.0000x reference)
import jax
import jax.numpy as jnp
from jax.experimental import pallas as pl


def kernel(center_ids, context_ids, neg_context_ids, emb_mu, emb_log_sigma, enc_emb, W_f, b_f, W_mu, b_mu, W_sig, b_sig):
    raise NotImplementedError("write your pallas kernel here")


# speedup vs baseline: 6.1619x; 6.1619x over previous
import functools

import jax
import jax.numpy as jnp
from jax import lax
from jax.experimental import pallas as pl
from jax.experimental.pallas import tpu as pltpu
from jax.experimental.pallas import tpu_sc as plsc

NC = 2
NS = 16
NW = NC * NS
CHUNK = 128


@functools.lru_cache(maxsize=None)
def _build_sc_gather(B, WIN, V, D):
    n_ctx = B * WIN
    b_per_tile = B // NW
    ctx_per_tile = n_ctx // NW
    f32 = jnp.float32

    mesh = plsc.VectorSubcoreMesh(
        core_axis_name="c", subcore_axis_name="s", num_cores=NC, num_subcores=NS
    )

    @functools.partial(
        pl.kernel,
        out_type=[
            jax.ShapeDtypeStruct((B, D), f32),
            jax.ShapeDtypeStruct((B, D), f32),
            jax.ShapeDtypeStruct((B,), f32),
            jax.ShapeDtypeStruct((n_ctx, D), f32),
            jax.ShapeDtypeStruct((n_ctx, D), f32),
            jax.ShapeDtypeStruct((n_ctx,), f32),
            jax.ShapeDtypeStruct((n_ctx, D), f32),
            jax.ShapeDtypeStruct((n_ctx,), f32),
        ],
        mesh=mesh,
        compiler_params=pltpu.CompilerParams(use_tc_tiling_on_sc=False),
        scratch_types=[
            pltpu.VMEM((CHUNK,), jnp.int32),
            pltpu.VMEM((CHUNK, D), f32),
            pltpu.VMEM((CHUNK, D), f32),
            pltpu.VMEM((CHUNK,), f32),
            pltpu.SemaphoreType.DMA,
            pltpu.SemaphoreType.DMA,
            pltpu.SemaphoreType.DMA,
        ],
    )
    def sc_gather(center_hbm, ctx_hbm, neg_hbm, emb_mu_hbm, lsig_hbm, enc_hbm,
                  enc_c_out, mu_c_out, ls_c_out, enc_ctx_out, mu_ctx_out,
                  ls_ctx_out, mu_neg_out, ls_neg_out,
                  idx_v, rows_enc, rows_mu, ls_v, sem_e, sem_m, sem_l):
        wid = lax.axis_index("s") * NC + lax.axis_index("c")

        def do_chunk(ids_hbm, base, enc_out, mu_out, ls_out):
            pltpu.sync_copy(ids_hbm.at[pl.ds(base, CHUNK)], idx_v)
            cp_e = None
            if enc_out is not None:
                cp_e = pltpu.async_copy(enc_hbm.at[idx_v], rows_enc, sem_e)
            cp_m = pltpu.async_copy(emb_mu_hbm.at[idx_v], rows_mu, sem_m)
            cp_l = pltpu.async_copy(lsig_hbm.at[idx_v], ls_v, sem_l)
            if cp_e is not None:
                cp_e.wait()
                pltpu.sync_copy(rows_enc, enc_out.at[pl.ds(base, CHUNK)])
            cp_m.wait()
            pltpu.sync_copy(rows_mu, mu_out.at[pl.ds(base, CHUNK)])
            cp_l.wait()
            pltpu.sync_copy(ls_v, ls_out.at[pl.ds(base, CHUNK)])

        b0 = wid * b_per_tile
        c0 = wid * ctx_per_tile

        @pl.loop(0, b_per_tile // CHUNK)
        def _center(k):
            do_chunk(center_hbm, b0 + k * CHUNK, enc_c_out, mu_c_out, ls_c_out)

        @pl.loop(0, ctx_per_tile // CHUNK)
        def _ctx(k):
            do_chunk(ctx_hbm, c0 + k * CHUNK, enc_ctx_out, mu_ctx_out, ls_ctx_out)

        @pl.loop(0, ctx_per_tile // CHUNK)
        def _neg(k):
            do_chunk(neg_hbm, c0 + k * CHUNK, None, mu_neg_out, ls_neg_out)

    return sc_gather


def _tc_body(WIN, LATENT, H,
             enc_c_ref, enc_ctx_ref, mu_c_ref, mu_ctx_ref, mu_neg_ref,
             ls_c_ref, ls_ctx_ref, ls_neg_ref,
             wf_ref, bf_ref, wmu_ref, bmu_ref, wsig_ref, bsig_ref,
             kl_ref, hinge_ref):
    f32 = jnp.float32
    d = float(LATENT)
    ec = enc_c_ref[...]
    wf = wf_ref[...]
    D = ec.shape[1]
    hc = jnp.dot(ec, wf[:D], preferred_element_type=f32) + bf_ref[...]
    ectx = enc_ctx_ref[...]
    h = jnp.zeros(hc.shape, f32)
    for w in range(WIN):
        hw = jnp.dot(ectx[:, w, :], wf[D:], preferred_element_type=f32)
        h = h + jnp.maximum(hc + hw, 0.0)
    mu_q = jnp.dot(h, wmu_ref[...], preferred_element_type=f32) + bmu_ref[...]
    sg = jnp.dot(h, wsig_ref[...], preferred_element_type=f32) + bsig_ref[...]
    sigma_q = (jnp.maximum(sg, 0.0)
               + jnp.log(1.0 + jnp.exp(-jnp.abs(sg))) + 1e-6)
    lsq = jnp.log(sigma_q)
    a_q = d * sigma_q * sigma_q

    def kl(sq, ls_t):
        return d * (ls_t - lsq) + (a_q + sq) * (0.5 * jnp.exp(-2.0 * ls_t)) - 0.5 * d

    dc = mu_q - mu_c_ref[...]
    sqc = jnp.sum(dc * dc, axis=1, keepdims=True)
    kl_part = jnp.sum(kl(sqc, ls_c_ref[...])).reshape(1, 1)

    mu_ctx = mu_ctx_ref[...]
    mu_neg = mu_neg_ref[...]
    ls_ctx = ls_ctx_ref[...]
    ls_neg = ls_neg_ref[...]
    hinge_part = jnp.zeros((1, 1), f32)
    for w in range(WIN):
        dp = mu_q - mu_ctx[:, w, :]
        dn = mu_q - mu_neg[:, w, :]
        sqp = jnp.sum(dp * dp, axis=1, keepdims=True)
        sqn = jnp.sum(dn * dn, axis=1, keepdims=True)
        klp = kl(sqp, ls_ctx[:, w:w + 1])
        kln = kl(sqn, ls_neg[:, w:w + 1])
        hinge_part = hinge_part + jnp.sum(jnp.maximum(klp - kln + 1.0, 0.0)).reshape(1, 1)

    i = pl.program_id(0)

    @pl.when(i == 0)
    def _():
        kl_ref[...] = kl_part
        hinge_ref[...] = hinge_part

    @pl.when(i > 0)
    def _():
        kl_ref[...] += kl_part
        hinge_ref[...] += hinge_part


@functools.lru_cache(maxsize=None)
def _build_tc(B, WIN, D, LATENT, H, Bb):
    f32 = jnp.float32
    grid = (B // Bb,)
    body = functools.partial(_tc_body, WIN, LATENT, H)
    const = lambda *shape: pl.BlockSpec(shape, lambda i: (0,) * len(shape))
    return pl.pallas_call(
        body,
        grid=grid,
        in_specs=[
            pl.BlockSpec((Bb, D), lambda i: (i, 0)),
            pl.BlockSpec((Bb, WIN, D), lambda i: (i, 0, 0)),
            pl.BlockSpec((Bb, LATENT), lambda i: (i, 0)),
            pl.BlockSpec((Bb, WIN, LATENT), lambda i: (i, 0, 0)),
            pl.BlockSpec((Bb, WIN, LATENT), lambda i: (i, 0, 0)),
            pl.BlockSpec((Bb, 1), lambda i: (i, 0)),
            pl.BlockSpec((Bb, WIN), lambda i: (i, 0)),
            pl.BlockSpec((Bb, WIN), lambda i: (i, 0)),
            const(2 * D, H),
            const(1, H),
            const(H, LATENT),
            const(1, LATENT),
            const(H, 1),
            const(1, 1),
        ],
        out_specs=[const(1, 1), const(1, 1)],
        out_shape=[
            jax.ShapeDtypeStruct((1, 1), f32),
            jax.ShapeDtypeStruct((1, 1), f32),
        ],
    )


def kernel(center_ids, context_ids, neg_context_ids, emb_mu, emb_log_sigma,
           enc_emb, W_f, b_f, W_mu, b_mu, W_sig, b_sig):
    B = center_ids.shape[0]
    WIN = context_ids.shape[1]
    V, LATENT = emb_mu.shape
    D = enc_emb.shape[1]
    H = W_f.shape[1]

    c_ids = center_ids.astype(jnp.int32)
    ctx_ids = context_ids.astype(jnp.int32).reshape(-1)
    neg_ids = neg_context_ids.astype(jnp.int32).reshape(-1)
    lsig = emb_log_sigma.reshape(-1)

    sc = _build_sc_gather(B, WIN, V, D)
    (enc_c, mu_c, ls_c, enc_ctx, mu_ctx, ls_ctx, mu_neg, ls_neg) = sc(
        c_ids, ctx_ids, neg_ids, emb_mu, lsig, enc_emb)

    Bb = 512
    tc = _build_tc(B, WIN, D, LATENT, H, Bb)
    kl_sum, hinge_sum = tc(
        enc_c, enc_ctx.reshape(B, WIN, D), mu_c,
        mu_ctx.reshape(B, WIN, LATENT), mu_neg.reshape(B, WIN, LATENT),
        ls_c.reshape(B, 1), ls_ctx.reshape(B, WIN), ls_neg.reshape(B, WIN),
        W_f, b_f.reshape(1, H), W_mu, b_mu.reshape(1, LATENT),
        W_sig, b_sig.reshape(1, 1))

    kl = kl_sum[0, 0] / B
    max_margin = hinge_sum[0, 0] / (B * WIN)
    return (kl, max_margin)


# speedup vs baseline: 7.6896x; 1.2479x over previous
import functools

import jax
import jax.numpy as jnp
from jax import lax
from jax.experimental import pallas as pl
from jax.experimental.pallas import tpu as pltpu
from jax.experimental.pallas import tpu_sc as plsc

NC = 2
NS = 16
NW = NC * NS
CHUNK = 128


@functools.lru_cache(maxsize=None)
def _build_sc_gather(B, WIN, V, D):
    n_ctx = B * WIN
    b_per_tile = B // NW
    ctx_per_tile = n_ctx // NW
    f32 = jnp.float32

    mesh = plsc.VectorSubcoreMesh(
        core_axis_name="c", subcore_axis_name="s", num_cores=NC, num_subcores=NS
    )

    @functools.partial(
        pl.kernel,
        out_type=[
            jax.ShapeDtypeStruct((B, D), f32),
            jax.ShapeDtypeStruct((B, D), f32),
            jax.ShapeDtypeStruct((B,), f32),
            jax.ShapeDtypeStruct((n_ctx, D), f32),
            jax.ShapeDtypeStruct((n_ctx, D), f32),
            jax.ShapeDtypeStruct((n_ctx,), f32),
            jax.ShapeDtypeStruct((n_ctx, D), f32),
            jax.ShapeDtypeStruct((n_ctx,), f32),
        ],
        mesh=mesh,
        compiler_params=pltpu.CompilerParams(use_tc_tiling_on_sc=False),
        scratch_types=[
            pltpu.VMEM((CHUNK,), jnp.int32),
            pltpu.VMEM((CHUNK, D), f32),
            pltpu.VMEM((CHUNK, D), f32),
            pltpu.VMEM((CHUNK,), f32),
            pltpu.SemaphoreType.DMA,
            pltpu.SemaphoreType.DMA,
            pltpu.SemaphoreType.DMA,
        ],
    )
    def sc_gather(center_hbm, ctx_hbm, neg_hbm, emb_mu_hbm, lsig_hbm, enc_hbm,
                  enc_c_out, mu_c_out, ls_c_out, enc_ctx_out, mu_ctx_out,
                  ls_ctx_out, mu_neg_out, ls_neg_out,
                  idx_v, rows_enc, rows_mu, ls_v, sem_e, sem_m, sem_l):
        wid = lax.axis_index("s") * NC + lax.axis_index("c")

        def do_chunk(ids_hbm, base, enc_out, mu_out, ls_out):
            pltpu.sync_copy(ids_hbm.at[pl.ds(base, CHUNK)], idx_v)
            cp_e = None
            if enc_out is not None:
                cp_e = pltpu.async_copy(enc_hbm.at[idx_v], rows_enc, sem_e)
            cp_m = pltpu.async_copy(emb_mu_hbm.at[idx_v], rows_mu, sem_m)
            cp_l = pltpu.async_copy(lsig_hbm.at[idx_v], ls_v, sem_l)
            if cp_e is not None:
                cp_e.wait()
                pltpu.sync_copy(rows_enc, enc_out.at[pl.ds(base, CHUNK)])
            cp_m.wait()
            pltpu.sync_copy(rows_mu, mu_out.at[pl.ds(base, CHUNK)])
            cp_l.wait()
            pltpu.sync_copy(ls_v, ls_out.at[pl.ds(base, CHUNK)])

        b0 = wid * b_per_tile
        c0 = wid * ctx_per_tile

        @pl.loop(0, b_per_tile // CHUNK)
        def _center(k):
            do_chunk(center_hbm, b0 + k * CHUNK, enc_c_out, mu_c_out, ls_c_out)

        @pl.loop(0, ctx_per_tile // CHUNK)
        def _ctx(k):
            do_chunk(ctx_hbm, c0 + k * CHUNK, enc_ctx_out, mu_ctx_out, ls_ctx_out)

        @pl.loop(0, ctx_per_tile // CHUNK)
        def _neg(k):
            do_chunk(neg_hbm, c0 + k * CHUNK, None, mu_neg_out, ls_neg_out)

    return sc_gather


def _tc_body(WIN, LATENT, H,
             enc_c_ref, enc_ctx_ref, mu_c_ref, mu_ctx_ref, mu_neg_ref,
             ls_c_ref, ls_ctx_ref, ls_neg_ref,
             wf_ref, bf_ref, wmu_ref, bmu_ref, wsig_ref, bsig_ref,
             kl_ref, hinge_ref):
    f32 = jnp.float32
    d = float(LATENT)
    ec = enc_c_ref[...]
    wf = wf_ref[...]
    D = ec.shape[1]
    hc = jnp.dot(ec, wf[:D], preferred_element_type=f32) + bf_ref[...]
    ectx = enc_ctx_ref[...]
    h = jnp.zeros(hc.shape, f32)
    for w in range(WIN):
        hw = jnp.dot(ectx[:, w, :], wf[D:], preferred_element_type=f32)
        h = h + jnp.maximum(hc + hw, 0.0)
    mu_q = jnp.dot(h, wmu_ref[...], preferred_element_type=f32) + bmu_ref[...]
    sg = jnp.dot(h, wsig_ref[...], preferred_element_type=f32) + bsig_ref[...]
    sigma_q = (jnp.maximum(sg, 0.0)
               + jnp.log(1.0 + jnp.exp(-jnp.abs(sg))) + 1e-6)
    lsq = jnp.log(sigma_q)
    a_q = d * sigma_q * sigma_q

    def kl(sq, ls_t):
        return d * (ls_t - lsq) + (a_q + sq) * (0.5 * jnp.exp(-2.0 * ls_t)) - 0.5 * d

    dc = mu_q - mu_c_ref[...]
    sqc = jnp.sum(dc * dc, axis=1, keepdims=True)
    kl_part = jnp.sum(kl(sqc, ls_c_ref[...])).reshape(1, 1)

    mu_q3 = mu_q[:, None, :]
    dp = mu_q3 - mu_ctx_ref[...]
    dn = mu_q3 - mu_neg_ref[...]
    sqp = jnp.sum(dp * dp, axis=2)
    sqn = jnp.sum(dn * dn, axis=2)
    klp = kl(sqp, ls_ctx_ref[...])
    kln = kl(sqn, ls_neg_ref[...])
    hinge_part = jnp.sum(jnp.maximum(klp - kln + 1.0, 0.0)).reshape(1, 1)

    i = pl.program_id(0)

    @pl.when(i == 0)
    def _():
        kl_ref[...] = kl_part
        hinge_ref[...] = hinge_part

    @pl.when(i > 0)
    def _():
        kl_ref[...] += kl_part
        hinge_ref[...] += hinge_part


@functools.lru_cache(maxsize=None)
def _build_tc(B, WIN, D, LATENT, H, Bb):
    f32 = jnp.float32
    grid = (B // Bb,)
    body = functools.partial(_tc_body, WIN, LATENT, H)
    const = lambda *shape: pl.BlockSpec(shape, lambda i: (0,) * len(shape))
    return pl.pallas_call(
        body,
        grid=grid,
        in_specs=[
            pl.BlockSpec((Bb, D), lambda i: (i, 0)),
            pl.BlockSpec((Bb, WIN, D), lambda i: (i, 0, 0)),
            pl.BlockSpec((Bb, LATENT), lambda i: (i, 0)),
            pl.BlockSpec((Bb, WIN, LATENT), lambda i: (i, 0, 0)),
            pl.BlockSpec((Bb, WIN, LATENT), lambda i: (i, 0, 0)),
            pl.BlockSpec((Bb, 1), lambda i: (i, 0)),
            pl.BlockSpec((Bb, WIN), lambda i: (i, 0)),
            pl.BlockSpec((Bb, WIN), lambda i: (i, 0)),
            const(2 * D, H),
            const(1, H),
            const(H, LATENT),
            const(1, LATENT),
            const(H, 1),
            const(1, 1),
        ],
        out_specs=[const(1, 1), const(1, 1)],
        out_shape=[
            jax.ShapeDtypeStruct((1, 1), f32),
            jax.ShapeDtypeStruct((1, 1), f32),
        ],
    )


def kernel(center_ids, context_ids, neg_context_ids, emb_mu, emb_log_sigma,
           enc_emb, W_f, b_f, W_mu, b_mu, W_sig, b_sig):
    B = center_ids.shape[0]
    WIN = context_ids.shape[1]
    V, LATENT = emb_mu.shape
    D = enc_emb.shape[1]
    H = W_f.shape[1]

    c_ids = center_ids.astype(jnp.int32)
    ctx_ids = context_ids.astype(jnp.int32).reshape(-1)
    neg_ids = neg_context_ids.astype(jnp.int32).reshape(-1)
    lsig = emb_log_sigma.reshape(-1)

    sc = _build_sc_gather(B, WIN, V, D)
    (enc_c, mu_c, ls_c, enc_ctx, mu_ctx, ls_ctx, mu_neg, ls_neg) = sc(
        c_ids, ctx_ids, neg_ids, emb_mu, lsig, enc_emb)

    Bb = 512
    tc = _build_tc(B, WIN, D, LATENT, H, Bb)
    kl_sum, hinge_sum = tc(
        enc_c, enc_ctx.reshape(B, WIN, D), mu_c,
        mu_ctx.reshape(B, WIN, LATENT), mu_neg.reshape(B, WIN, LATENT),
        ls_c.reshape(B, 1), ls_ctx.reshape(B, WIN), ls_neg.reshape(B, WIN),
        W_f, b_f.reshape(1, H), W_mu, b_mu.reshape(1, LATENT),
        W_sig, b_sig.reshape(1, 1))

    kl = kl_sum[0, 0] / B
    max_margin = hinge_sum[0, 0] / (B * WIN)
    return (kl, max_margin)
